# baseline (speedup 1.0000x reference)
import functools

import jax
import jax.numpy as jnp
from jax import lax
from jax.experimental import pallas as pl
from jax.experimental.pallas import tpu as pltpu
from jax.experimental.pallas import tpu_sc as plsc

_NC = 2
_NS = 16
_L = 16


def _proj_body(h_ref, wl_ref, wr_ref, xl_ref, xr_ref):
    hblk = h_ref[...]
    xl_ref[0] = jnp.dot(hblk, wl_ref[0], preferred_element_type=jnp.float32)
    xr_ref[0] = jnp.dot(hblk, wr_ref[0], preferred_element_type=jnp.float32)


def _proj(h, Wl, Wr, H):
    N, D = h.shape
    BN = 400
    G = N // BN
    wl3 = Wl.reshape(D, H, D).transpose(1, 0, 2)
    wr3 = Wr.reshape(D, H, D).transpose(1, 0, 2)
    return pl.pallas_call(
        _proj_body,
        grid=(H, G),
        in_specs=[
            pl.BlockSpec((BN, D), lambda hh, i: (i, 0)),
            pl.BlockSpec((1, D, D), lambda hh, i: (hh, 0, 0)),
            pl.BlockSpec((1, D, D), lambda hh, i: (hh, 0, 0)),
        ],
        out_specs=[
            pl.BlockSpec((1, BN, D), lambda hh, i: (hh, i, 0)),
            pl.BlockSpec((1, BN, D), lambda hh, i: (hh, i, 0)),
        ],
        out_shape=[
            jax.ShapeDtypeStruct((H, N, D), jnp.float32),
            jax.ShapeDtypeStruct((H, N, D), jnp.float32),
        ],
    )(h, wl3, wr3)


def _denred_body(den_ref, out_ref):
    out_ref[0] = jnp.sum(den_ref[0], axis=0)[:, None]


def _denred(den):
    H, NS, NP = den.shape
    BD = 128
    return pl.pallas_call(
        _denred_body,
        grid=(H, NP // BD),
        in_specs=[pl.BlockSpec((1, NS, BD), lambda h, i: (h, 0, i))],
        out_specs=pl.BlockSpec((1, BD, 1), lambda h, i: (h, i, 0)),
        out_shape=jax.ShapeDtypeStruct((H, NP, 1), jnp.float32),
    )(den)


def _combine_body(num_ref, den_ref, b_ref, hin_ref, out_ref, *, H, relu):
    num = num_ref[...]
    den = den_ref[...]
    o = jnp.sum(num / (den + 1e-16), axis=0) * (1.0 / H)
    o = o + b_ref[0] + hin_ref[...]
    if relu:
        o = jnp.maximum(o, 0.0)
    out_ref[...] = o


def _combine(num, den, b, hin, relu):
    H = num.shape[0]
    N, D = hin.shape
    BN = 400
    G = N // BN
    return pl.pallas_call(
        functools.partial(_combine_body, H=H, relu=relu),
        grid=(G,),
        in_specs=[
            pl.BlockSpec((H, BN, D), lambda i: (0, i, 0)),
            pl.BlockSpec((H, BN, 1), lambda i: (0, i, 0)),
            pl.BlockSpec((1, D), lambda i: (0, 0)),
            pl.BlockSpec((BN, D), lambda i: (i, 0)),
        ],
        out_specs=pl.BlockSpec((BN, D), lambda i: (i, 0)),
        out_shape=jax.ShapeDtypeStruct((N, D), jnp.float32),
    )(num, den, b.reshape(1, D), hin)


def _edge_pass(xl, xr, cidx, att):
    H, N, D = xl.shape
    NCHT, _, K = cidx.shape
    HC = H // _NC
    NCH = NCHT // _NS
    NB = 3
    NP = N + 112
    RT = NP // _NS
    NJ = D // _L
    KP = K + 8

    mesh = plsc.VectorSubcoreMesh(core_axis_name="c", subcore_axis_name="s")

    @functools.partial(
        pl.kernel,
        out_type=[
            jax.ShapeDtypeStruct((H, NP, D), jnp.float32),
            jax.ShapeDtypeStruct((H, _NS, NP), jnp.float32),
        ],
        mesh=mesh,
        compiler_params=pltpu.CompilerParams(needs_layout_passes=False,
                                             use_tc_tiling_on_sc=False),
        scratch_types=[
            pltpu.VMEM((NB, 2, KP), jnp.int32),
            pltpu.VMEM((NB, K, D), jnp.float32),
            pltpu.VMEM((NB, K, D), jnp.float32),
            pltpu.VMEM((K, D), jnp.float32),
            pltpu.VMEM((1, K), jnp.int32),
            pltpu.VMEM((_L, D), jnp.float32),
            pltpu.VMEM((NP,), jnp.float32),
            pltpu.VMEM((D,), jnp.float32),
            pltpu.VMEM((K * _L,), jnp.float32),
            pltpu.VMEM_SHARED((NP, D), jnp.float32),
            [pltpu.SemaphoreType.DMA] * NB,
            [pltpu.SemaphoreType.DMA] * NB,
            [pltpu.SemaphoreType.DMA] * NB,
            pltpu.SemaphoreType.DMA,
        ],
    )
    def ek(xl_ref, xr_ref, cidx_ref, att_ref, num_ref, den_ref,
           cib, rl, rr, stg, sci, zbuf, dent, attv, abuf, acc,
           semi, seml, semr, sems):
        c = lax.axis_index("c")
        s = lax.axis_index("s")
        zv = jnp.zeros((_L,), jnp.float32)
        lane_iota = jnp.arange(_L, dtype=jnp.int32)

        def zrow(i, _):
            for j in range(NJ):
                zbuf[i, pl.ds(j * _L, _L)] = zv
            return 0

        lax.fori_loop(0, _L, zrow, 0)

        def idx_cp(j, b):
            return pltpu.make_async_copy(
                cidx_ref.at[s * NCH + j],
                cib.at[b, :, pl.ds(0, K)], semi[b])

        def gl_cp(h, b):
            return pltpu.make_async_copy(
                xl_ref.at[h].at[cib.at[b, 0, pl.ds(0, K)]],
                rl.at[b], seml[b])

        def gr_cp(h, b):
            return pltpu.make_async_copy(
                xr_ref.at[h].at[cib.at[b, 1, pl.ds(0, K)]],
                rr.at[b], semr[b])

        def sc_cp():
            return pltpu.make_async_copy(stg, acc.at[sci.at[0]], sems)

        def compute_chunk(b, dvs, atr):
            def e1(i, _):
                aacc = zv
                for j in range(NJ):
                    t = (rl[b, i, pl.ds(j * _L, _L)]
                         + rr[b, i, pl.ds(j * _L, _L)])
                    t = jnp.maximum(t, 0.2 * t)
                    aacc = aacc + t * atr[j]
                abuf[pl.ds(i * _L, _L)] = aacc
                return 0

            lax.fori_loop(0, K, e1, 0)

            for gi, (g0, n) in enumerate(
                    ((0, _L), (_L, _L), (2 * _L, K - 2 * _L))):
                colbase = (g0 + lane_iota) * _L
                alphas = plsc.load_gather(abuf, [colbase])
                for cc in range(1, _L):
                    alphas = alphas + plsc.load_gather(abuf, [colbase + cc])
                exs = jnp.exp(jnp.minimum(alphas, 50.0))
                dvec = dvs[gi]
                for k in range(n):
                    i = g0 + k
                    ex = jnp.full((_L,), exs[k], jnp.float32)
                    for j in range(NJ):
                        stg[i, pl.ds(j * _L, _L)] = (
                            ex * rl[b, i, pl.ds(j * _L, _L)])
                    d_e = dvec[k]
                    dbase = (d_e >> 4) << 4
                    lane = d_e - dbase
                    cur = dent[pl.ds(dbase, _L)]
                    dent[pl.ds(dbase, _L)] = cur + jnp.where(
                        lane_iota == lane, ex, 0.0)
            return atr

        def head_body(hh, _):
            h = hh * _NC + c

            done = 0
            while done < RT:
                n = min(_L, RT - done)
                pltpu.sync_copy(zbuf.at[pl.ds(0, n), :],
                                acc.at[pl.ds(s * RT + done, n), :])
                done += n

            def zden(i, _):
                dent[pl.ds(i * _L, _L)] = zv
                return 0

            lax.fori_loop(0, NP // _L, zden, 0)
            plsc.subcore_barrier()

            pltpu.sync_copy(att_ref.at[h], attv)
            att_regs = tuple(attv[pl.ds(j * _L, _L)] for j in range(NJ))

            for b in range(NB):
                idx_cp(b, b).start()
            for b in range(NB - 1):
                idx_cp(b, b).wait()
                gl_cp(h, b).start()
                gr_cp(h, b).start()

            def outer(jj, atr):
                for b in range(NB):
                    j = NB * jj + b
                    bn = (b + NB - 1) % NB
                    gl_cp(h, b).wait()
                    gr_cp(h, b).wait()
                    @pl.when(j >= 1)
                    def _():
                        sc_cp().wait()
                    dvs = (cib[b, 1, pl.ds(0, _L)],
                           cib[b, 1, pl.ds(_L, _L)],
                           cib[b, 1, pl.ds(2 * _L, _L)])
                    sci[0, pl.ds(0, _L)] = dvs[0]
                    sci[0, pl.ds(_L, _L)] = dvs[1]
                    sci[0, pl.ds(K - _L, _L)] = cib[b, 1, pl.ds(K - _L, _L)]
                    @pl.when(j + NB < NCH)
                    def _():
                        idx_cp(j + NB, b).start()
                    @pl.when(j + NB - 1 < NCH)
                    def _():
                        idx_cp(j + NB - 1, bn).wait()
                        gl_cp(h, bn).start()
                        gr_cp(h, bn).start()
                    atr = compute_chunk(b, dvs, atr)
                    pltpu.async_copy(stg, acc.at[sci.at[0]], sems, add=True)
                return atr

            lax.fori_loop(0, NCH // NB, outer, att_regs)
            sc_cp().wait()
            plsc.subcore_barrier()

            pltpu.sync_copy(acc.at[pl.ds(s * RT, RT), :],
                            num_ref.at[h].at[pl.ds(s * RT, RT), :])
            pltpu.sync_copy(dent, den_ref.at[h].at[s])
            plsc.subcore_barrier()
            return 0

        lax.fori_loop(0, HC, head_body, 0)

    return ek(xl, xr, cidx, att)


def _gat_layer(h, cidx, Wl, Wr, att, b, H, relu):
    xl, xr = _proj(h, Wl, Wr, H)
    num, den = _edge_pass(xl, xr, cidx, att)
    return _combine(num, _denred(den), b, h, relu)


def kernel(x, edge_index, Wl1, Wr1, att1, b1, Wl2, Wr2, att2, b2,
           Wl3, Wr3, att3, b3):
    K = 40
    N = x.shape[0]
    cidx = jnp.stack([edge_index[0].reshape(-1, K),
                      edge_index[1].reshape(-1, K)], axis=1)
    nt = cidx.shape[0] // _NS
    ntp = ((nt + 2) // 3) * 3
    dummy = jnp.concatenate(
        [jnp.zeros((_NS, ntp - nt, 1, K), jnp.int32),
         jnp.full((_NS, ntp - nt, 1, K), N, jnp.int32)], axis=2)
    cidx = jnp.concatenate(
        [cidx.reshape(_NS, nt, 2, K), dummy], axis=1).reshape(-1, 2, K)
    h = x
    h = _gat_layer(h, cidx, Wl1, Wr1, att1, b1, 8, True)
    h = _gat_layer(h, cidx, Wl2, Wr2, att2, b2, 8, True)
    h = _gat_layer(h, cidx, Wl3, Wr3, att3, b3, 4, False)
    return h


# speedup vs baseline: 1.3821x; 1.3821x over previous
import functools

import jax
import jax.numpy as jnp
from jax import lax
from jax.experimental import pallas as pl
from jax.experimental.pallas import tpu as pltpu
from jax.experimental.pallas import tpu_sc as plsc

_NC = 2
_NS = 16
_L = 16


def _proj_body(h_ref, wl_ref, wr_ref, xl_ref, xr_ref):
    hblk = h_ref[...]
    xl_ref[0] = jnp.dot(hblk, wl_ref[0], preferred_element_type=jnp.float32)
    xr_ref[0] = jnp.dot(hblk, wr_ref[0], preferred_element_type=jnp.float32)


def _proj(h, Wl, Wr, H):
    N, D = h.shape
    BN = 400
    G = N // BN
    wl3 = Wl.reshape(D, H, D).transpose(1, 0, 2)
    wr3 = Wr.reshape(D, H, D).transpose(1, 0, 2)
    return pl.pallas_call(
        _proj_body,
        grid=(H, G),
        in_specs=[
            pl.BlockSpec((BN, D), lambda hh, i: (i, 0)),
            pl.BlockSpec((1, D, D), lambda hh, i: (hh, 0, 0)),
            pl.BlockSpec((1, D, D), lambda hh, i: (hh, 0, 0)),
        ],
        out_specs=[
            pl.BlockSpec((1, BN, D), lambda hh, i: (hh, i, 0)),
            pl.BlockSpec((1, BN, D), lambda hh, i: (hh, i, 0)),
        ],
        out_shape=[
            jax.ShapeDtypeStruct((H, N, D), jnp.float32),
            jax.ShapeDtypeStruct((H, N, D), jnp.float32),
        ],
    )(h, wl3, wr3)


def _denred_body(den_ref, out_ref):
    out_ref[0] = jnp.sum(den_ref[0], axis=0)[:, None]


def _denred(den):
    H, NS, NP = den.shape
    BD = 128
    return pl.pallas_call(
        _denred_body,
        grid=(H, NP // BD),
        in_specs=[pl.BlockSpec((1, NS, BD), lambda h, i: (h, 0, i))],
        out_specs=pl.BlockSpec((1, BD, 1), lambda h, i: (h, i, 0)),
        out_shape=jax.ShapeDtypeStruct((H, NP, 1), jnp.float32),
    )(den)


def _combine_body(num_ref, den_ref, b_ref, hin_ref, out_ref, *, H, relu):
    num = num_ref[...]
    den = den_ref[...]
    o = jnp.sum(num / (den + 1e-16), axis=0) * (1.0 / H)
    o = o + b_ref[0] + hin_ref[...]
    if relu:
        o = jnp.maximum(o, 0.0)
    out_ref[...] = o


def _combine(num, den, b, hin, relu):
    H = num.shape[0]
    N, D = hin.shape
    BN = 400
    G = N // BN
    return pl.pallas_call(
        functools.partial(_combine_body, H=H, relu=relu),
        grid=(G,),
        in_specs=[
            pl.BlockSpec((H, BN, D), lambda i: (0, i, 0)),
            pl.BlockSpec((H, BN, 1), lambda i: (0, i, 0)),
            pl.BlockSpec((1, D), lambda i: (0, 0)),
            pl.BlockSpec((BN, D), lambda i: (i, 0)),
        ],
        out_specs=pl.BlockSpec((BN, D), lambda i: (i, 0)),
        out_shape=jax.ShapeDtypeStruct((N, D), jnp.float32),
    )(num, den, b.reshape(1, D), hin)


def _edge_pass(xl, xr, cidx, att):
    H, N, D = xl.shape
    NCHT, _, KP = cidx.shape
    K = KP - 8
    HC = H // _NC
    NCH = NCHT // _NS
    NB = 3
    NP = N + 112
    RT = NP // _NS
    NJ = D // _L

    mesh = plsc.VectorSubcoreMesh(core_axis_name="c", subcore_axis_name="s")

    @functools.partial(
        pl.kernel,
        out_type=[
            jax.ShapeDtypeStruct((H, NP, D), jnp.float32),
            jax.ShapeDtypeStruct((H, _NS, NP), jnp.float32),
        ],
        mesh=mesh,
        compiler_params=pltpu.CompilerParams(needs_layout_passes=False,
                                             use_tc_tiling_on_sc=False),
        scratch_types=[
            pltpu.VMEM((NB, 2, KP), jnp.int32),
            pltpu.VMEM((NB, K, D), jnp.float32),
            pltpu.VMEM((NB, K, D), jnp.float32),
            pltpu.VMEM((K, D), jnp.float32),
            pltpu.VMEM((1, K), jnp.int32),
            pltpu.VMEM((_L, D), jnp.float32),
            pltpu.VMEM((NP,), jnp.float32),
            pltpu.VMEM((D,), jnp.float32),
            pltpu.VMEM((K * _L,), jnp.float32),
            pltpu.VMEM_SHARED((NP, D), jnp.float32),
            [pltpu.SemaphoreType.DMA] * NB,
            [pltpu.SemaphoreType.DMA] * NB,
            [pltpu.SemaphoreType.DMA] * NB,
            pltpu.SemaphoreType.DMA,
        ],
    )
    def ek(xl_ref, xr_ref, cidx_ref, att_ref, num_ref, den_ref,
           cib, rl, rr, stg, sci, zbuf, dent, attv, abuf, acc,
           semi, seml, semr, sems):
        c = lax.axis_index("c")
        s = lax.axis_index("s")
        zv = jnp.zeros((_L,), jnp.float32)
        lane_iota = jnp.arange(_L, dtype=jnp.int32)

        def zrow(i, _):
            for j in range(NJ):
                zbuf[i, pl.ds(j * _L, _L)] = zv
            return 0

        lax.fori_loop(0, _L, zrow, 0)

        def idx_cp(j, b):
            return pltpu.make_async_copy(
                cidx_ref.at[s * NCH + j], cib.at[b], semi[b])

        def gl_cp(h, b):
            return pltpu.make_async_copy(
                xl_ref.at[h].at[cib.at[b, 0, pl.ds(0, K)]],
                rl.at[b], seml[b])

        def gr_cp(h, b):
            return pltpu.make_async_copy(
                xr_ref.at[h].at[cib.at[b, 1, pl.ds(0, K)]],
                rr.at[b], semr[b])

        def sc_cp():
            return pltpu.make_async_copy(stg, acc.at[sci.at[0]], sems)

        def compute_chunk(b, dvs, flags, atr):
            def e1(i, _):
                aacc = zv
                for j in range(NJ):
                    t = (rl[b, i, pl.ds(j * _L, _L)]
                         + rr[b, i, pl.ds(j * _L, _L)])
                    t = jnp.maximum(t, 0.2 * t)
                    aacc = aacc + t * atr[j]
                abuf[pl.ds(i * _L, _L)] = aacc
                return 0

            lax.fori_loop(0, K, e1, 0)

            for gi in range(K // _L):
                g0 = gi * _L
                colbase = (g0 + lane_iota) * _L
                cols = [plsc.load_gather(abuf, [colbase + cc])
                        for cc in range(_L)]
                while len(cols) > 1:
                    cols = [cols[2 * q] + cols[2 * q + 1]
                            for q in range(len(cols) // 2)]
                exs = jnp.exp(jnp.minimum(cols[0], 50.0))
                dvec = dvs[gi]

                @pl.when(flags[8 + gi] == 0)
                def _():
                    plsc.addupdate_scatter(dent, [dvec], exs)

                @pl.when(flags[8 + gi] != 0)
                def _():
                    for k in range(_L):
                        d_e = dvec[k]
                        dbase = (d_e >> 4) << 4
                        lane = d_e - dbase
                        ex = jnp.full((_L,), exs[k], jnp.float32)
                        cur = dent[pl.ds(dbase, _L)]
                        dent[pl.ds(dbase, _L)] = cur + jnp.where(
                            lane_iota == lane, ex, 0.0)

                for k in range(_L):
                    i = g0 + k
                    ex = jnp.full((_L,), exs[k], jnp.float32)
                    for j in range(NJ):
                        stg[i, pl.ds(j * _L, _L)] = (
                            ex * rl[b, i, pl.ds(j * _L, _L)])
            return atr

        def head_body(hh, _):
            h = hh * _NC + c

            done = 0
            while done < RT:
                n = min(_L, RT - done)
                pltpu.sync_copy(zbuf.at[pl.ds(0, n), :],
                                acc.at[pl.ds(s * RT + done, n), :])
                done += n

            def zden(i, _):
                dent[pl.ds(i * _L, _L)] = zv
                return 0

            lax.fori_loop(0, NP // _L, zden, 0)
            plsc.subcore_barrier()

            pltpu.sync_copy(att_ref.at[h], attv)
            att_regs = tuple(attv[pl.ds(j * _L, _L)] for j in range(NJ))

            for b in range(NB):
                idx_cp(b, b).start()
            for b in range(NB - 1):
                idx_cp(b, b).wait()
                gl_cp(h, b).start()
                gr_cp(h, b).start()

            def outer(jj, atr):
                for b in range(NB):
                    j = NB * jj + b
                    bn = (b + NB - 1) % NB
                    gl_cp(h, b).wait()
                    gr_cp(h, b).wait()
                    @pl.when(j >= 1)
                    def _():
                        sc_cp().wait()
                    dvs = (cib[b, 1, pl.ds(0, _L)],
                           cib[b, 1, pl.ds(_L, _L)])
                    sci[0, pl.ds(0, _L)] = dvs[0]
                    sci[0, pl.ds(_L, _L)] = dvs[1]
                    fv = cib[b, 1, pl.ds(KP - _L, _L)]
                    @pl.when(j + NB < NCH)
                    def _():
                        idx_cp(j + NB, b).start()
                    @pl.when(j + NB - 1 < NCH)
                    def _():
                        idx_cp(j + NB - 1, bn).wait()
                        gl_cp(h, bn).start()
                        gr_cp(h, bn).start()
                    atr = compute_chunk(b, dvs, fv, atr)
                    pltpu.async_copy(stg, acc.at[sci.at[0]], sems, add=True)
                return atr

            lax.fori_loop(0, NCH // NB, outer, att_regs)
            sc_cp().wait()
            plsc.subcore_barrier()

            pltpu.sync_copy(acc.at[pl.ds(s * RT, RT), :],
                            num_ref.at[h].at[pl.ds(s * RT, RT), :])
            pltpu.sync_copy(dent, den_ref.at[h].at[s])
            plsc.subcore_barrier()
            return 0

        lax.fori_loop(0, HC, head_body, 0)

    return ek(xl, xr, cidx, att)


def _gat_layer(h, cidx, Wl, Wr, att, b, H, relu):
    xl, xr = _proj(h, Wl, Wr, H)
    num, den = _edge_pass(xl, xr, cidx, att)
    return _combine(num, _denred(den), b, h, relu)


def kernel(x, edge_index, Wl1, Wr1, att1, b1, Wl2, Wr2, att2, b2,
           Wl3, Wr3, att3, b3):
    K = 32
    N = x.shape[0]
    src2 = edge_index[0].reshape(-1, K)
    dst2 = edge_index[1].reshape(-1, K)
    nch0 = src2.shape[0]

    def dupflag(g):
        eq = (g[:, :, None] == g[:, None, :]).sum((1, 2))
        return (eq > _L).astype(jnp.int32)[:, None]

    zpad = jnp.zeros((nch0, 8), jnp.int32)
    srcp = jnp.concatenate([src2, zpad], axis=1)
    dstp = jnp.concatenate(
        [dst2, dupflag(dst2[:, :_L]), dupflag(dst2[:, _L:]),
         zpad[:, :6]], axis=1)
    cidx = jnp.stack([srcp, dstp], axis=1)
    nt = nch0 // _NS
    ntp = ((nt + 2) // 3) * 3
    dsrc = jnp.zeros((_NS, ntp - nt, 1, 40), jnp.int32)
    lanes = jnp.arange(40, dtype=jnp.int32)
    ddst = jnp.broadcast_to(jnp.where(lanes < K, N + lanes % _L, 0),
                            (_NS, ntp - nt, 1, 40))
    cidx = jnp.concatenate(
        [cidx.reshape(_NS, nt, 2, 40),
         jnp.concatenate([dsrc, ddst], axis=2)], axis=1).reshape(-1, 2, 40)
    h = x
    h = _gat_layer(h, cidx, Wl1, Wr1, att1, b1, 8, True)
    h = _gat_layer(h, cidx, Wl2, Wr2, att2, b2, 8, True)
    h = _gat_layer(h, cidx, Wl3, Wr3, att3, b3, 4, False)
    return h
